# Initial kernel scaffold; baseline (speedup 1.0000x reference)
import jax
import jax.numpy as jnp
from jax.experimental import pallas as pl


def kernel(x, edges, W1, b1, g1, be1, W2, b2, g2, be2, W3, b3, g3, be3, gW1, gas1, gad1, gb1, g4, be4, gW2, gas2, gad2, gb2, glg, glbe, pW1, pb1, pg1, pbe1, pW2, pb2, pg2, pbe2, pW3, pb3):
    raise NotImplementedError("write your pallas kernel here")


# speedup vs baseline: 1.0384x; 1.0384x over previous
import functools
import jax
import jax.numpy as jnp
from jax.experimental import pallas as pl
from jax.experimental.pallas import tpu as pltpu

N = 10000
HEADS = 2
OUT = 256
HID = HEADS * OUT
ROWS = 1000
GRID = N // ROWS
EPS = 1e-5


def _elu(x):
    return jnp.where(x > 0, x, jnp.exp(x) - 1.0)


def _stats_rows(y):
    s = jnp.concatenate([y.sum(0, keepdims=True), (y * y).sum(0, keepdims=True)], 0)
    return jnp.concatenate([s, jnp.zeros((6, y.shape[1]), jnp.float32)], 0)


def _lin_stats_body(x_ref, w_ref, b_ref, y_ref, s_ref):
    y = jnp.dot(x_ref[...], w_ref[...], preferred_element_type=jnp.float32) + b_ref[...]
    y_ref[...] = y

    @pl.when(pl.program_id(0) == 0)
    def _():
        s_ref[...] = jnp.zeros_like(s_ref)

    s_ref[...] += _stats_rows(y)


def _bn_lin_body(y_ref, s_ref, g_ref, be_ref, w_ref, b_ref, o_ref, so_ref):
    m = s_ref[0:1, :] / N
    v = s_ref[1:2, :] / N - m * m
    z = (y_ref[...] - m) * jax.lax.rsqrt(v + EPS) * g_ref[...] + be_ref[...]
    z = _elu(z)
    o = jnp.dot(z, w_ref[...], preferred_element_type=jnp.float32) + b_ref[...]
    o_ref[...] = o

    @pl.when(pl.program_id(0) == 0)
    def _():
        so_ref[...] = jnp.zeros_like(so_ref)

    so_ref[...] += _stats_rows(o)


def _bn_lin_aux_body(y_ref, s_ref, g_ref, be_ref, w_ref, as_ref, ad_ref,
                     o_ref, aux_ref):
    m = s_ref[0:1, :] / N
    v = s_ref[1:2, :] / N - m * m
    z = (y_ref[...] - m) * jax.lax.rsqrt(v + EPS) * g_ref[...] + be_ref[...]
    z = _elu(z)
    o = jnp.dot(z, w_ref[...], preferred_element_type=jnp.float32)
    o_ref[...] = o
    oh = o.reshape(ROWS, HEADS, OUT)
    als = (oh * as_ref[...]).sum(-1)
    ald = (oh * ad_ref[...]).sum(-1)
    aux = jnp.concatenate([als, ald, jnp.zeros((ROWS, 4), jnp.float32)], 1)
    aux_ref[...] = aux


def _post_gat_body(acc_ref, den_ref, gb_ref, t_ref, s_ref):
    den = den_ref[...]
    denw = jnp.broadcast_to(den[:, :, None], (ROWS, HEADS, OUT)).reshape(ROWS, HID)
    t = acc_ref[...] / (denw + 1e-16) + gb_ref[...]
    t_ref[...] = t

    @pl.when(pl.program_id(0) == 0)
    def _():
        s_ref[...] = jnp.zeros_like(s_ref)

    s_ref[...] += _stats_rows(t)


def _final_body(y_ref, s_ref, g_ref, be_ref, w_ref, b_ref, o_ref):
    m = s_ref[0:1, :] / N
    v = s_ref[1:2, :] / N - m * m
    z = (y_ref[...] - m) * jax.lax.rsqrt(v + EPS) * g_ref[...] + be_ref[...]
    z = _elu(z)
    o_ref[...] = jnp.dot(z, w_ref[...], preferred_element_type=jnp.float32) + b_ref[...]


def _row_spec(c):
    return pl.BlockSpec((ROWS, c), lambda i: (i, 0))


def _full_spec(shape):
    nd = len(shape)
    return pl.BlockSpec(shape, lambda i: (0,) * nd)


def _lin_stats(x, w, b):
    cin, cout = w.shape
    return pl.pallas_call(
        _lin_stats_body,
        grid=(GRID,),
        in_specs=[_row_spec(cin), _full_spec(w.shape), _full_spec((1, cout))],
        out_specs=[_row_spec(cout), _full_spec((8, cout))],
        out_shape=[jax.ShapeDtypeStruct((N, cout), jnp.float32),
                   jax.ShapeDtypeStruct((8, cout), jnp.float32)],
    )(x, w, b.reshape(1, cout))


def _bn_lin(y, s, g, be, w, b):
    cin, cout = w.shape
    return pl.pallas_call(
        _bn_lin_body,
        grid=(GRID,),
        in_specs=[_row_spec(cin), _full_spec((8, cin)), _full_spec((1, cin)),
                  _full_spec((1, cin)), _full_spec(w.shape), _full_spec((1, cout))],
        out_specs=[_row_spec(cout), _full_spec((8, cout))],
        out_shape=[jax.ShapeDtypeStruct((N, cout), jnp.float32),
                   jax.ShapeDtypeStruct((8, cout), jnp.float32)],
    )(y, s, g.reshape(1, cin), be.reshape(1, cin), w, b.reshape(1, cout))


def _bn_lin_aux(y, s, g, be, w, a_s, a_d):
    cin, cout = w.shape
    return pl.pallas_call(
        _bn_lin_aux_body,
        grid=(GRID,),
        in_specs=[_row_spec(cin), _full_spec((8, cin)), _full_spec((1, cin)),
                  _full_spec((1, cin)), _full_spec(w.shape),
                  _full_spec((1, HEADS, OUT)), _full_spec((1, HEADS, OUT))],
        out_specs=[_row_spec(cout), _row_spec(8)],
        out_shape=[jax.ShapeDtypeStruct((N, cout), jnp.float32),
                   jax.ShapeDtypeStruct((N, 8), jnp.float32)],
    )(y, s, g.reshape(1, cin), be.reshape(1, cin), w,
      a_s.reshape(1, HEADS, OUT), a_d.reshape(1, HEADS, OUT))


def _post_gat(acc, den, gb):
    return pl.pallas_call(
        _post_gat_body,
        grid=(GRID,),
        in_specs=[_row_spec(HID), _row_spec(2), _full_spec((1, HID))],
        out_specs=[_row_spec(HID), _full_spec((8, HID))],
        out_shape=[jax.ShapeDtypeStruct((N, HID), jnp.float32),
                   jax.ShapeDtypeStruct((8, HID), jnp.float32)],
    )(acc, den, gb.reshape(1, HID))


def _final(y, s, g, be, w, b):
    cin, cout = w.shape
    return pl.pallas_call(
        _final_body,
        grid=(GRID,),
        in_specs=[_row_spec(cin), _full_spec((8, cin)), _full_spec((1, cin)),
                  _full_spec((1, cin)), _full_spec(w.shape), _full_spec((1, cout))],
        out_specs=_row_spec(cout),
        out_shape=jax.ShapeDtypeStruct((N, cout), jnp.float32),
    )(y, s, g.reshape(1, cin), be.reshape(1, cin), w, b.reshape(1, cout))


def _edge_phase(hp, aux, src, dst):
    als = aux[:, 0:2]
    ald = aux[:, 2:4]
    e = als[src] + ald[dst]
    e = jnp.where(e > 0, e, 0.2 * e)
    ee = jnp.exp(e)
    den = jax.ops.segment_sum(ee, dst, num_segments=N)
    h3 = hp.reshape(N, HEADS, OUT)
    acc = jax.ops.segment_sum(h3[src] * ee[:, :, None], dst, num_segments=N)
    return acc.reshape(N, HID), den


def kernel(x, edges, W1, b1, g1, be1, W2, b2, g2, be2, W3, b3, g3, be3,
           gW1, gas1, gad1, gb1, g4, be4, gW2, gas2, gad2, gb2, glg, glbe,
           pW1, pb1, pg1, pbe1, pW2, pb2, pg2, pbe2, pW3, pb3):
    loop = jnp.arange(N, dtype=edges.dtype)
    src = jnp.concatenate([edges[0], loop])
    dst = jnp.concatenate([edges[1], loop])

    y1, s1 = _lin_stats(x, W1, b1)
    y2, s2 = _bn_lin(y1, s1, g1, be1, W2, b2)
    y3, s3 = _bn_lin(y2, s2, g2, be2, W3, b3)
    hp1, aux1 = _bn_lin_aux(y3, s3, g3, be3, gW1, gas1, gad1)
    acc1, den1 = _edge_phase(hp1, aux1, src, dst)
    t1, st1 = _post_gat(acc1, den1, gb1)
    hp2, aux2 = _bn_lin_aux(t1, st1, g4, be4, gW2, gas2, gad2)
    acc2, den2 = _edge_phase(hp2, aux2, src, dst)
    t2, st2 = _post_gat(acc2, den2, gb2)
    z1, sz1 = _bn_lin(t2, st2, glg, glbe, pW1, pb1)
    z2, sz2 = _bn_lin(z1, sz1, pg1, pbe1, pW2, pb2)
    return _final(z2, sz2, pg2, pbe2, pW3, pb3)


# speedup vs baseline: 7.1125x; 6.8498x over previous
import functools
import jax
import jax.numpy as jnp
from jax import lax
from jax.experimental import pallas as pl
from jax.experimental.pallas import tpu as pltpu
from jax.experimental.pallas import tpu_sc as plsc

N = 10000
HEADS = 2
OUT = 256
HID = HEADS * OUT
ROWS = 1000
GRID = N // ROWS
EPS = 1e-5

NP = 10016
E = 160000 + N
NW = 32
EP = 171008
EC = EP // NW
STEPS = EC // 16
AUXW = NP * 4
DENW = NP * 2
RNG = 160
NRNG = 64
NOUT = RNG * NRNG
SB = 2672
NSB = EP // SB
PCAP = 2080
PTH = 2048
ZR = RNG


def _elu(x):
    return jnp.where(x > 0, x, jnp.exp(x) - 1.0)


def _stats_rows(y):
    s = jnp.concatenate([y.sum(0, keepdims=True), (y * y).sum(0, keepdims=True)], 0)
    return jnp.concatenate([s, jnp.zeros((6, y.shape[1]), jnp.float32)], 0)


def _lin_stats_body(x_ref, w_ref, b_ref, y_ref, s_ref):
    y = jnp.dot(x_ref[...], w_ref[...], preferred_element_type=jnp.float32) + b_ref[...]
    y_ref[...] = y

    @pl.when(pl.program_id(0) == 0)
    def _():
        s_ref[...] = jnp.zeros_like(s_ref)

    s_ref[...] += _stats_rows(y)


def _bn_lin_body(y_ref, s_ref, g_ref, be_ref, w_ref, b_ref, o_ref, so_ref):
    m = s_ref[0:1, :] / N
    v = s_ref[1:2, :] / N - m * m
    z = (y_ref[...] - m) * jax.lax.rsqrt(v + EPS) * g_ref[...] + be_ref[...]
    z = _elu(z)
    o = jnp.dot(z, w_ref[...], preferred_element_type=jnp.float32) + b_ref[...]
    o_ref[...] = o

    @pl.when(pl.program_id(0) == 0)
    def _():
        so_ref[...] = jnp.zeros_like(so_ref)

    so_ref[...] += _stats_rows(o)


def _bn_lin_aux_body(y_ref, s_ref, g_ref, be_ref, w_ref, as_ref, ad_ref,
                     o_ref, aux_ref):
    m = s_ref[0:1, :] / N
    v = s_ref[1:2, :] / N - m * m
    z = (y_ref[...] - m) * jax.lax.rsqrt(v + EPS) * g_ref[...] + be_ref[...]
    z = _elu(z)
    o = jnp.dot(z, w_ref[...], preferred_element_type=jnp.float32)
    o_ref[...] = o
    oh = o.reshape(ROWS, HEADS, OUT)
    als = (oh * as_ref[...]).sum(-1)
    ald = (oh * ad_ref[...]).sum(-1)
    aux = jnp.concatenate([als, ald, jnp.zeros((ROWS, 4), jnp.float32)], 1)
    aux_ref[...] = aux


def _den_reduce_body(dp_ref, out_ref):
    out_ref[...] = dp_ref[...].sum(0, keepdims=True)


def _den_reduce(den_part):
    return pl.pallas_call(
        _den_reduce_body,
        in_specs=[pl.BlockSpec((NW, DENW), lambda: (0, 0))],
        out_specs=pl.BlockSpec((1, DENW), lambda: (0, 0)),
        out_shape=jax.ShapeDtypeStruct((1, DENW), jnp.float32),
    )(den_part)


def _post_gat_body(acc_ref, gb_ref, t_ref, s_ref):
    t = acc_ref[...] + gb_ref[...]
    t_ref[...] = t

    @pl.when(pl.program_id(0) == 0)
    def _():
        s_ref[...] = jnp.zeros_like(s_ref)

    s_ref[...] += _stats_rows(t)


def _final_body(y_ref, s_ref, g_ref, be_ref, w_ref, b_ref, o_ref):
    m = s_ref[0:1, :] / N
    v = s_ref[1:2, :] / N - m * m
    z = (y_ref[...] - m) * jax.lax.rsqrt(v + EPS) * g_ref[...] + be_ref[...]
    z = _elu(z)
    o_ref[...] = jnp.dot(z, w_ref[...], preferred_element_type=jnp.float32) + b_ref[...]


def _row_spec(c):
    return pl.BlockSpec((ROWS, c), lambda i: (i, 0))


def _full_spec(shape):
    nd = len(shape)
    return pl.BlockSpec(shape, lambda i: (0,) * nd)


def _lin_stats(x, w, b):
    cin, cout = w.shape
    return pl.pallas_call(
        _lin_stats_body,
        grid=(GRID,),
        in_specs=[_row_spec(cin), _full_spec(w.shape), _full_spec((1, cout))],
        out_specs=[_row_spec(cout), _full_spec((8, cout))],
        out_shape=[jax.ShapeDtypeStruct((N, cout), jnp.float32),
                   jax.ShapeDtypeStruct((8, cout), jnp.float32)],
    )(x, w, b.reshape(1, cout))


def _bn_lin(y, s, g, be, w, b):
    cin, cout = w.shape
    return pl.pallas_call(
        _bn_lin_body,
        grid=(GRID,),
        in_specs=[_row_spec(cin), _full_spec((8, cin)), _full_spec((1, cin)),
                  _full_spec((1, cin)), _full_spec(w.shape), _full_spec((1, cout))],
        out_specs=[_row_spec(cout), _full_spec((8, cout))],
        out_shape=[jax.ShapeDtypeStruct((N, cout), jnp.float32),
                   jax.ShapeDtypeStruct((8, cout), jnp.float32)],
    )(y, s, g.reshape(1, cin), be.reshape(1, cin), w, b.reshape(1, cout))


def _bn_lin_aux(y, s, g, be, w, a_s, a_d):
    cin, cout = w.shape
    return pl.pallas_call(
        _bn_lin_aux_body,
        grid=(GRID,),
        in_specs=[_row_spec(cin), _full_spec((8, cin)), _full_spec((1, cin)),
                  _full_spec((1, cin)), _full_spec(w.shape),
                  _full_spec((1, HEADS, OUT)), _full_spec((1, HEADS, OUT))],
        out_specs=[_row_spec(cout), _row_spec(8)],
        out_shape=[jax.ShapeDtypeStruct((N, cout), jnp.float32),
                   jax.ShapeDtypeStruct((N, 8), jnp.float32)],
    )(y, s, g.reshape(1, cin), be.reshape(1, cin), w,
      a_s.reshape(1, HEADS, OUT), a_d.reshape(1, HEADS, OUT))


def _post_gat(acc, gb):
    return pl.pallas_call(
        _post_gat_body,
        grid=(GRID,),
        in_specs=[pl.BlockSpec((ROWS, HID), lambda i: (i, 0)),
                  _full_spec((1, HID))],
        out_specs=[_row_spec(HID), _full_spec((8, HID))],
        out_shape=[jax.ShapeDtypeStruct((N, HID), jnp.float32),
                   jax.ShapeDtypeStruct((8, HID), jnp.float32)],
    )(acc, gb.reshape(1, HID))


def _final(y, s, g, be, w, b):
    cin, cout = w.shape
    return pl.pallas_call(
        _final_body,
        grid=(GRID,),
        in_specs=[_row_spec(cin), _full_spec((8, cin)), _full_spec((1, cin)),
                  _full_spec((1, cin)), _full_spec(w.shape), _full_spec((1, cout))],
        out_specs=_row_spec(cout),
        out_shape=jax.ShapeDtypeStruct((N, cout), jnp.float32),
    )(y, s, g.reshape(1, cin), be.reshape(1, cin), w, b.reshape(1, cout))


_MESH = dict(core_axis_name="c", subcore_axis_name="s")


_SC_PARAMS = pltpu.CompilerParams(needs_layout_passes=False)


def _sc_scalar_body(auxp_h, srcp_h, dstp_h, zden_h, ee0_h, ee1_h, den_h,
                    aux_v, src_v, dst_v, ee0_v, ee1_v, den_v):
    c = lax.axis_index("c")
    s = lax.axis_index("s")
    w = c * 16 + s
    base = w * EC
    pltpu.sync_copy(auxp_h, aux_v)
    pltpu.sync_copy(srcp_h.at[pl.ds(base, EC)], src_v)
    pltpu.sync_copy(dstp_h.at[pl.ds(base, EC)], dst_v)
    pltpu.sync_copy(zden_h, den_v)

    def step(i, carry):
        sv = src_v[pl.ds(i * 16, 16)]
        dv = dst_v[pl.ds(i * 16, 16)]
        a0 = plsc.load_gather(aux_v, [sv * 4])
        a1 = plsc.load_gather(aux_v, [sv * 4 + 1])
        b0 = plsc.load_gather(aux_v, [dv * 4 + 2])
        b1 = plsc.load_gather(aux_v, [dv * 4 + 3])
        e0 = a0 + b0
        e0 = jnp.where(e0 > 0, e0, 0.2 * e0)
        x0 = jnp.exp(e0)
        e1 = a1 + b1
        e1 = jnp.where(e1 > 0, e1, 0.2 * e1)
        x1 = jnp.exp(e1)
        ee0_v[pl.ds(i * 16, 16)] = x0
        ee1_v[pl.ds(i * 16, 16)] = x1
        plsc.addupdate_scatter(den_v, [dv * 2], x0)
        plsc.addupdate_scatter(den_v, [dv * 2 + 1], x1)
        return carry

    lax.fori_loop(0, STEPS, step, 0)
    pltpu.sync_copy(ee0_v, ee0_h.at[pl.ds(base, EC)])
    pltpu.sync_copy(ee1_v, ee1_h.at[pl.ds(base, EC)])
    pltpu.sync_copy(den_v, den_h.at[w])


def _sc_scalar(auxp, srcp, dstp, zden):
    return pl.kernel(
        _sc_scalar_body,
        out_type=[jax.ShapeDtypeStruct((EP,), jnp.float32),
                  jax.ShapeDtypeStruct((EP,), jnp.float32),
                  jax.ShapeDtypeStruct((NW, DENW), jnp.float32)],
        mesh=plsc.VectorSubcoreMesh(**_MESH),
        scratch_types=[pltpu.VMEM((AUXW,), jnp.float32),
                       pltpu.VMEM((EC,), jnp.int32),
                       pltpu.VMEM((EC,), jnp.int32),
                       pltpu.VMEM((EC,), jnp.float32),
                       pltpu.VMEM((EC,), jnp.float32),
                       pltpu.VMEM((DENW,), jnp.float32)],
        compiler_params=_SC_PARAMS,
    )(auxp, srcp, dstp, zden)


def _sc_alpha_body(ee0_h, ee1_h, dstp_h, den_h, a0_h, a1_h,
                   dst_v, ee0_v, ee1_v, den_v):
    c = lax.axis_index("c")
    s = lax.axis_index("s")
    base = (c * 16 + s) * EC
    pltpu.sync_copy(den_h, den_v)
    pltpu.sync_copy(dstp_h.at[pl.ds(base, EC)], dst_v)
    pltpu.sync_copy(ee0_h.at[pl.ds(base, EC)], ee0_v)
    pltpu.sync_copy(ee1_h.at[pl.ds(base, EC)], ee1_v)

    def step(i, carry):
        dv = dst_v[pl.ds(i * 16, 16)]
        d0 = plsc.load_gather(den_v, [dv * 2])
        d1 = plsc.load_gather(den_v, [dv * 2 + 1])
        ee0_v[pl.ds(i * 16, 16)] = ee0_v[pl.ds(i * 16, 16)] / (d0 + 1e-16)
        ee1_v[pl.ds(i * 16, 16)] = ee1_v[pl.ds(i * 16, 16)] / (d1 + 1e-16)
        return carry

    lax.fori_loop(0, STEPS, step, 0)
    pltpu.sync_copy(ee0_v, a0_h.at[pl.ds(base, EC)])
    pltpu.sync_copy(ee1_v, a1_h.at[pl.ds(base, EC)])


def _sc_alpha(ee0, ee1, dstp, den):
    return pl.kernel(
        _sc_alpha_body,
        out_type=[jax.ShapeDtypeStruct((EP,), jnp.float32),
                  jax.ShapeDtypeStruct((EP,), jnp.float32)],
        mesh=plsc.VectorSubcoreMesh(**_MESH),
        scratch_types=[pltpu.VMEM((EC,), jnp.int32),
                       pltpu.VMEM((EC,), jnp.float32),
                       pltpu.VMEM((EC,), jnp.float32),
                       pltpu.VMEM((DENW,), jnp.float32)],
        compiler_params=_SC_PARAMS,
    )(ee0, ee1, dstp, den)


def _sc_vector_body(hp_h, srcp_h, dstp_h, a0_h, a1_h, z_h, out_h,
                    eb, relb, srcb, a0b, a1b, buf, acc, sem):
    c = lax.axis_index("c")
    s = lax.axis_index("s")
    w = c * 16 + s

    def dgroup(g, carry):
        srcv = srcb[pl.ds(g * 16, 16)]
        pltpu.async_copy(hp_h.at[srcv], buf, sem).wait()
        relv = relb[pl.ds(g * 16, 16)]
        for j in range(16):
            ij = g * 16 + j
            jj = jnp.full((16,), ij, jnp.int32)
            s0 = plsc.load_gather(a0b, [jj])
            s1 = plsc.load_gather(a1b, [jj])
            relj = relv[j]

            def qstep(qq, cr):
                sc = jnp.where(qq < 2, s0, s1)
                for t in range(8):
                    sl = pl.ds(qq * 128 + t * 16, 16)
                    acc[relj, sl] = acc[relj, sl] + buf[j, sl] * sc
                return cr

            lax.fori_loop(0, 4, qstep, 0)
        return carry

    def drain(na):
        k = na // 16
        lax.fori_loop(0, k, dgroup, 0)
        r = na - k * 16
        rv = relb[pl.ds(k * 16, 16)]
        sv = srcb[pl.ds(k * 16, 16)]
        av = a0b[pl.ds(k * 16, 16)]
        bv = a1b[pl.ds(k * 16, 16)]
        relb[pl.ds(0, 16)] = rv
        srcb[pl.ds(0, 16)] = sv
        a0b[pl.ds(0, 16)] = av
        a1b[pl.ds(0, 16)] = bv
        return r

    def pbody(p, carry):
        rng = p * 32 + w
        rbase = rng * RNG
        pltpu.sync_copy(z_h, acc.at[pl.ds(0, RNG)])

        def sstep(i, na):
            sv = plsc.bitcast(eb[pl.ds(i * 16, 16)], jnp.int32)
            dv = plsc.bitcast(eb[pl.ds(SB + i * 16, 16)], jnp.int32)
            x0 = eb[pl.ds(2 * SB + i * 16, 16)]
            x1 = eb[pl.ds(3 * SB + i * 16, 16)]
            rel = dv - rbase
            m = (rel >= 0) & (rel < RNG)
            plsc.store_compressed(relb.at[pl.ds(na, 16)], rel, mask=m)
            plsc.store_compressed(srcb.at[pl.ds(na, 16)], sv, mask=m)
            plsc.store_compressed(a0b.at[pl.ds(na, 16)], x0, mask=m)
            plsc.store_compressed(a1b.at[pl.ds(na, 16)], x1, mask=m)
            cnt = jnp.max(plsc.all_reduce_population_count(m))
            na = na + cnt
            return lax.cond(na >= PTH, drain, lambda v: v, na)

        def scan_block(b, na):
            off = b * SB
            pltpu.sync_copy(srcp_h.at[pl.ds(off, SB)], eb.at[pl.ds(0, SB)])
            pltpu.sync_copy(dstp_h.at[pl.ds(off, SB)], eb.at[pl.ds(SB, SB)])
            pltpu.sync_copy(a0_h.at[pl.ds(off, SB)], eb.at[pl.ds(2 * SB, SB)])
            pltpu.sync_copy(a1_h.at[pl.ds(off, SB)], eb.at[pl.ds(3 * SB, SB)])
            return lax.fori_loop(0, SB // 16, sstep, na)

        na = lax.fori_loop(0, NSB, scan_block, 0)
        relb[pl.ds(na, 16)] = jnp.full((16,), RNG, jnp.int32)
        srcb[pl.ds(na, 16)] = jnp.zeros((16,), jnp.int32)
        a0b[pl.ds(na, 16)] = jnp.zeros((16,), jnp.float32)
        a1b[pl.ds(na, 16)] = jnp.zeros((16,), jnp.float32)
        lax.fori_loop(0, (na + 15) // 16, dgroup, 0)
        pltpu.sync_copy(acc.at[pl.ds(0, RNG)],
                        out_h.at[pl.ds(rbase, RNG)])
        return carry

    lax.fori_loop(0, 2, pbody, 0)


def _sc_vector(hp_pad, srcf, dstf, a0, a1, zrows):
    return pl.kernel(
        _sc_vector_body,
        out_type=jax.ShapeDtypeStruct((NOUT, HID), jnp.float32),
        mesh=plsc.VectorSubcoreMesh(**_MESH),
        scratch_types=[pltpu.VMEM((4 * SB,), jnp.float32),
                       pltpu.VMEM((PCAP,), jnp.int32),
                       pltpu.VMEM((PCAP,), jnp.int32),
                       pltpu.VMEM((PCAP,), jnp.float32),
                       pltpu.VMEM((PCAP,), jnp.float32),
                       pltpu.VMEM((16, HID), jnp.float32),
                       pltpu.VMEM((RNG + 1, HID), jnp.float32),
                       pltpu.SemaphoreType.DMA],
        compiler_params=_SC_PARAMS,
    )(hp_pad, srcf, dstf, a0, a1, zrows)


def _edge_phase(hp, aux, srcp, dstp, zrows):
    auxp = jnp.pad(aux[:, :4], ((0, NP - N), (0, 0))).reshape(-1)
    hp_pad = jnp.pad(hp, ((0, NP - hp.shape[0]), (0, 0)))
    zden = jnp.zeros((DENW,), jnp.float32)
    ee0, ee1, den_part = _sc_scalar(auxp, srcp, dstp, zden)
    den = _den_reduce(den_part).reshape(-1)
    a0, a1 = _sc_alpha(ee0, ee1, dstp, den)
    srcf = lax.bitcast_convert_type(srcp, jnp.float32)
    dstf = lax.bitcast_convert_type(dstp, jnp.float32)
    return _sc_vector(hp_pad, srcf, dstf, a0, a1, zrows)


def kernel(x, edges, W1, b1, g1, be1, W2, b2, g2, be2, W3, b3, g3, be3,
           gW1, gas1, gad1, gb1, g4, be4, gW2, gas2, gad2, gb2, glg, glbe,
           pW1, pb1, pg1, pbe1, pW2, pb2, pg2, pbe2, pW3, pb3):
    loop = jnp.arange(N, dtype=edges.dtype)
    pad = jnp.full((EP - E,), N, dtype=edges.dtype)
    srcp = jnp.concatenate([edges[0], loop, pad])
    dstp = jnp.concatenate([edges[1], loop, pad])
    zrows = jnp.zeros((ZR, HID), jnp.float32)

    y1, s1 = _lin_stats(x, W1, b1)
    y2, s2 = _bn_lin(y1, s1, g1, be1, W2, b2)
    y3, s3 = _bn_lin(y2, s2, g2, be2, W3, b3)
    hp1, aux1 = _bn_lin_aux(y3, s3, g3, be3, gW1, gas1, gad1)
    acc1 = _edge_phase(hp1, aux1, srcp, dstp, zrows)
    t1, st1 = _post_gat(acc1, gb1)
    hp2, aux2 = _bn_lin_aux(t1, st1, g4, be4, gW2, gas2, gad2)
    acc2 = _edge_phase(hp2, aux2, srcp, dstp, zrows)
    t2, st2 = _post_gat(acc2, gb2)
    z1, sz1 = _bn_lin(t2, st2, glg, glbe, pW1, pb1)
    z2, sz2 = _bn_lin(z1, sz1, pg1, pbe1, pW2, pb2)
    return _final(z2, sz2, pg2, pbe2, pW3, pb3)


# speedup vs baseline: 7.5476x; 1.0612x over previous
import functools
import jax
import jax.numpy as jnp
from jax import lax
from jax.experimental import pallas as pl
from jax.experimental.pallas import tpu as pltpu
from jax.experimental.pallas import tpu_sc as plsc

N = 10000
HEADS = 2
OUT = 256
HID = HEADS * OUT
ROWS = 1000
GRID = N // ROWS
EPS = 1e-5

NP = 10016
E = 160000 + N
NW = 32
EP = 171008
EC = EP // NW
STEPS = EC // 16
AUXW = NP * 4
DENW = NP * 2
RNG = 160
NRNG = 64
NOUT = RNG * NRNG
SB = 2672
NSB = EP // SB
PCAP = 2112
PTH = 2048
ZR = RNG


def _elu(x):
    return jnp.where(x > 0, x, jnp.exp(x) - 1.0)


def _stats_rows(y):
    s = jnp.concatenate([y.sum(0, keepdims=True), (y * y).sum(0, keepdims=True)], 0)
    return jnp.concatenate([s, jnp.zeros((6, y.shape[1]), jnp.float32)], 0)


def _lin_stats_body(x_ref, w_ref, b_ref, y_ref, s_ref):
    y = jnp.dot(x_ref[...], w_ref[...], preferred_element_type=jnp.float32) + b_ref[...]
    y_ref[...] = y

    @pl.when(pl.program_id(0) == 0)
    def _():
        s_ref[...] = jnp.zeros_like(s_ref)

    s_ref[...] += _stats_rows(y)


def _bn_lin_body(y_ref, s_ref, g_ref, be_ref, w_ref, b_ref, o_ref, so_ref):
    m = s_ref[0:1, :] / N
    v = s_ref[1:2, :] / N - m * m
    z = (y_ref[...] - m) * jax.lax.rsqrt(v + EPS) * g_ref[...] + be_ref[...]
    z = _elu(z)
    o = jnp.dot(z, w_ref[...], preferred_element_type=jnp.float32) + b_ref[...]
    o_ref[...] = o

    @pl.when(pl.program_id(0) == 0)
    def _():
        so_ref[...] = jnp.zeros_like(so_ref)

    so_ref[...] += _stats_rows(o)


def _bn_lin_aux_body(y_ref, s_ref, g_ref, be_ref, w_ref, as_ref, ad_ref,
                     o_ref, aux_ref):
    m = s_ref[0:1, :] / N
    v = s_ref[1:2, :] / N - m * m
    z = (y_ref[...] - m) * jax.lax.rsqrt(v + EPS) * g_ref[...] + be_ref[...]
    z = _elu(z)
    o = jnp.dot(z, w_ref[...], preferred_element_type=jnp.float32)
    o_ref[...] = o
    oh = o.reshape(ROWS, HEADS, OUT)
    als = (oh * as_ref[...]).sum(-1)
    ald = (oh * ad_ref[...]).sum(-1)
    aux = jnp.concatenate([als, ald, jnp.zeros((ROWS, 4), jnp.float32)], 1)
    aux_ref[...] = aux


def _den_reduce_body(dp_ref, out_ref):
    out_ref[...] = dp_ref[...].sum(0, keepdims=True)


def _den_reduce(den_part):
    return pl.pallas_call(
        _den_reduce_body,
        in_specs=[pl.BlockSpec((NW, DENW), lambda: (0, 0))],
        out_specs=pl.BlockSpec((1, DENW), lambda: (0, 0)),
        out_shape=jax.ShapeDtypeStruct((1, DENW), jnp.float32),
    )(den_part)


def _post_gat_body(acc_ref, gb_ref, t_ref, s_ref):
    t = acc_ref[...] + gb_ref[...]
    t_ref[...] = t

    @pl.when(pl.program_id(0) == 0)
    def _():
        s_ref[...] = jnp.zeros_like(s_ref)

    s_ref[...] += _stats_rows(t)


def _final_body(y_ref, s_ref, g_ref, be_ref, w_ref, b_ref, o_ref):
    m = s_ref[0:1, :] / N
    v = s_ref[1:2, :] / N - m * m
    z = (y_ref[...] - m) * jax.lax.rsqrt(v + EPS) * g_ref[...] + be_ref[...]
    z = _elu(z)
    o_ref[...] = jnp.dot(z, w_ref[...], preferred_element_type=jnp.float32) + b_ref[...]


def _row_spec(c):
    return pl.BlockSpec((ROWS, c), lambda i: (i, 0))


def _full_spec(shape):
    nd = len(shape)
    return pl.BlockSpec(shape, lambda i: (0,) * nd)


def _lin_stats(x, w, b):
    cin, cout = w.shape
    return pl.pallas_call(
        _lin_stats_body,
        grid=(GRID,),
        in_specs=[_row_spec(cin), _full_spec(w.shape), _full_spec((1, cout))],
        out_specs=[_row_spec(cout), _full_spec((8, cout))],
        out_shape=[jax.ShapeDtypeStruct((N, cout), jnp.float32),
                   jax.ShapeDtypeStruct((8, cout), jnp.float32)],
    )(x, w, b.reshape(1, cout))


def _bn_lin(y, s, g, be, w, b):
    cin, cout = w.shape
    return pl.pallas_call(
        _bn_lin_body,
        grid=(GRID,),
        in_specs=[_row_spec(cin), _full_spec((8, cin)), _full_spec((1, cin)),
                  _full_spec((1, cin)), _full_spec(w.shape), _full_spec((1, cout))],
        out_specs=[_row_spec(cout), _full_spec((8, cout))],
        out_shape=[jax.ShapeDtypeStruct((N, cout), jnp.float32),
                   jax.ShapeDtypeStruct((8, cout), jnp.float32)],
    )(y, s, g.reshape(1, cin), be.reshape(1, cin), w, b.reshape(1, cout))


def _bn_lin_aux(y, s, g, be, w, a_s, a_d):
    cin, cout = w.shape
    return pl.pallas_call(
        _bn_lin_aux_body,
        grid=(GRID,),
        in_specs=[_row_spec(cin), _full_spec((8, cin)), _full_spec((1, cin)),
                  _full_spec((1, cin)), _full_spec(w.shape),
                  _full_spec((1, HEADS, OUT)), _full_spec((1, HEADS, OUT))],
        out_specs=[_row_spec(cout), _row_spec(8)],
        out_shape=[jax.ShapeDtypeStruct((N, cout), jnp.float32),
                   jax.ShapeDtypeStruct((N, 8), jnp.float32)],
    )(y, s, g.reshape(1, cin), be.reshape(1, cin), w,
      a_s.reshape(1, HEADS, OUT), a_d.reshape(1, HEADS, OUT))


def _post_gat(acc, gb):
    return pl.pallas_call(
        _post_gat_body,
        grid=(GRID,),
        in_specs=[pl.BlockSpec((ROWS, HID), lambda i: (i, 0)),
                  _full_spec((1, HID))],
        out_specs=[_row_spec(HID), _full_spec((8, HID))],
        out_shape=[jax.ShapeDtypeStruct((N, HID), jnp.float32),
                   jax.ShapeDtypeStruct((8, HID), jnp.float32)],
    )(acc, gb.reshape(1, HID))


def _final(y, s, g, be, w, b):
    cin, cout = w.shape
    return pl.pallas_call(
        _final_body,
        grid=(GRID,),
        in_specs=[_row_spec(cin), _full_spec((8, cin)), _full_spec((1, cin)),
                  _full_spec((1, cin)), _full_spec(w.shape), _full_spec((1, cout))],
        out_specs=_row_spec(cout),
        out_shape=jax.ShapeDtypeStruct((N, cout), jnp.float32),
    )(y, s, g.reshape(1, cin), be.reshape(1, cin), w, b.reshape(1, cout))


_MESH = dict(core_axis_name="c", subcore_axis_name="s")


_SC_PARAMS = pltpu.CompilerParams(needs_layout_passes=False)


def _sc_scalar_body(auxp_h, srcp_h, dstp_h, zden_h, ee0_h, ee1_h, den_h,
                    aux_v, src_v, dst_v, ee0_v, ee1_v, den_v):
    c = lax.axis_index("c")
    s = lax.axis_index("s")
    w = c * 16 + s
    base = w * EC
    pltpu.sync_copy(auxp_h, aux_v)
    pltpu.sync_copy(srcp_h.at[pl.ds(base, EC)], src_v)
    pltpu.sync_copy(dstp_h.at[pl.ds(base, EC)], dst_v)
    pltpu.sync_copy(zden_h, den_v)

    def step(i, carry):
        sv = src_v[pl.ds(i * 16, 16)]
        dv = dst_v[pl.ds(i * 16, 16)]
        a0 = plsc.load_gather(aux_v, [sv * 4])
        a1 = plsc.load_gather(aux_v, [sv * 4 + 1])
        b0 = plsc.load_gather(aux_v, [dv * 4 + 2])
        b1 = plsc.load_gather(aux_v, [dv * 4 + 3])
        e0 = a0 + b0
        e0 = jnp.where(e0 > 0, e0, 0.2 * e0)
        x0 = jnp.exp(e0)
        e1 = a1 + b1
        e1 = jnp.where(e1 > 0, e1, 0.2 * e1)
        x1 = jnp.exp(e1)
        ee0_v[pl.ds(i * 16, 16)] = x0
        ee1_v[pl.ds(i * 16, 16)] = x1
        plsc.addupdate_scatter(den_v, [dv * 2], x0)
        plsc.addupdate_scatter(den_v, [dv * 2 + 1], x1)
        return carry

    lax.fori_loop(0, STEPS, step, 0)
    pltpu.sync_copy(ee0_v, ee0_h.at[pl.ds(base, EC)])
    pltpu.sync_copy(ee1_v, ee1_h.at[pl.ds(base, EC)])
    pltpu.sync_copy(den_v, den_h.at[w])


def _sc_scalar(auxp, srcp, dstp, zden):
    return pl.kernel(
        _sc_scalar_body,
        out_type=[jax.ShapeDtypeStruct((EP,), jnp.float32),
                  jax.ShapeDtypeStruct((EP,), jnp.float32),
                  jax.ShapeDtypeStruct((NW, DENW), jnp.float32)],
        mesh=plsc.VectorSubcoreMesh(**_MESH),
        scratch_types=[pltpu.VMEM((AUXW,), jnp.float32),
                       pltpu.VMEM((EC,), jnp.int32),
                       pltpu.VMEM((EC,), jnp.int32),
                       pltpu.VMEM((EC,), jnp.float32),
                       pltpu.VMEM((EC,), jnp.float32),
                       pltpu.VMEM((DENW,), jnp.float32)],
        compiler_params=_SC_PARAMS,
    )(auxp, srcp, dstp, zden)


def _sc_alpha_body(ee0_h, ee1_h, dstp_h, den_h, a0_h, a1_h,
                   dst_v, ee0_v, ee1_v, den_v):
    c = lax.axis_index("c")
    s = lax.axis_index("s")
    base = (c * 16 + s) * EC
    pltpu.sync_copy(den_h, den_v)
    pltpu.sync_copy(dstp_h.at[pl.ds(base, EC)], dst_v)
    pltpu.sync_copy(ee0_h.at[pl.ds(base, EC)], ee0_v)
    pltpu.sync_copy(ee1_h.at[pl.ds(base, EC)], ee1_v)

    def step(i, carry):
        dv = dst_v[pl.ds(i * 16, 16)]
        d0 = plsc.load_gather(den_v, [dv * 2])
        d1 = plsc.load_gather(den_v, [dv * 2 + 1])
        ee0_v[pl.ds(i * 16, 16)] = ee0_v[pl.ds(i * 16, 16)] / (d0 + 1e-16)
        ee1_v[pl.ds(i * 16, 16)] = ee1_v[pl.ds(i * 16, 16)] / (d1 + 1e-16)
        return carry

    lax.fori_loop(0, STEPS, step, 0)
    pltpu.sync_copy(ee0_v, a0_h.at[pl.ds(base, EC)])
    pltpu.sync_copy(ee1_v, a1_h.at[pl.ds(base, EC)])


def _sc_alpha(ee0, ee1, dstp, den):
    return pl.kernel(
        _sc_alpha_body,
        out_type=[jax.ShapeDtypeStruct((EP,), jnp.float32),
                  jax.ShapeDtypeStruct((EP,), jnp.float32)],
        mesh=plsc.VectorSubcoreMesh(**_MESH),
        scratch_types=[pltpu.VMEM((EC,), jnp.int32),
                       pltpu.VMEM((EC,), jnp.float32),
                       pltpu.VMEM((EC,), jnp.float32),
                       pltpu.VMEM((DENW,), jnp.float32)],
        compiler_params=_SC_PARAMS,
    )(ee0, ee1, dstp, den)


def _sc_vector_body(hp_h, srcp_h, dstp_h, a0_h, a1_h, z_h, out_h,
                    eb, relb, srcb, a0b, a1b, buf, acc, sem):
    c = lax.axis_index("c")
    s = lax.axis_index("s")
    w = c * 16 + s

    def dgroup(g, carry):
        pltpu.async_copy(hp_h.at[srcb.at[pl.ds(g * 32, 32)]], buf, sem).wait()

        def hbody(half, cr):
            relv = relb[pl.ds(g * 32 + half * 16, 16)]
            for j in range(16):
                ij = g * 32 + half * 16 + j
                jj = jnp.full((16,), ij, jnp.int32)
                s0 = plsc.load_gather(a0b, [jj])
                s1 = plsc.load_gather(a1b, [jj])
                relj = relv[j]
                jb = half * 16 + j

                def qstep(qq, cr2):
                    sc = jnp.where(qq < 2, s0, s1)
                    for t in range(8):
                        sl = pl.ds(qq * 128 + t * 16, 16)
                        acc[relj, sl] = acc[relj, sl] + buf[jb, sl] * sc
                    return cr2

                lax.fori_loop(0, 4, qstep, 0)
            return cr

        lax.fori_loop(0, 2, hbody, 0)
        return carry

    def drain(na):
        k = na // 32
        lax.fori_loop(0, k, dgroup, 0)
        r = na - k * 32
        rv = relb[pl.ds(k * 32, 16)]
        sv = srcb[pl.ds(k * 32, 16)]
        av = a0b[pl.ds(k * 32, 16)]
        bv = a1b[pl.ds(k * 32, 16)]
        rv2 = relb[pl.ds(k * 32 + 16, 16)]
        sv2 = srcb[pl.ds(k * 32 + 16, 16)]
        av2 = a0b[pl.ds(k * 32 + 16, 16)]
        bv2 = a1b[pl.ds(k * 32 + 16, 16)]
        relb[pl.ds(0, 16)] = rv
        srcb[pl.ds(0, 16)] = sv
        a0b[pl.ds(0, 16)] = av
        a1b[pl.ds(0, 16)] = bv
        relb[pl.ds(16, 16)] = rv2
        srcb[pl.ds(16, 16)] = sv2
        a0b[pl.ds(16, 16)] = av2
        a1b[pl.ds(16, 16)] = bv2
        return r

    def pbody(p, carry):
        rng = p * 32 + w
        rbase = rng * RNG
        pltpu.sync_copy(z_h, acc.at[pl.ds(0, RNG)])

        def sstep(i, na):
            sv = plsc.bitcast(eb[pl.ds(i * 16, 16)], jnp.int32)
            dv = plsc.bitcast(eb[pl.ds(SB + i * 16, 16)], jnp.int32)
            x0 = eb[pl.ds(2 * SB + i * 16, 16)]
            x1 = eb[pl.ds(3 * SB + i * 16, 16)]
            rel = dv - rbase
            m = (rel >= 0) & (rel < RNG)
            plsc.store_compressed(relb.at[pl.ds(na, 16)], rel, mask=m)
            plsc.store_compressed(srcb.at[pl.ds(na, 16)], sv, mask=m)
            plsc.store_compressed(a0b.at[pl.ds(na, 16)], x0, mask=m)
            plsc.store_compressed(a1b.at[pl.ds(na, 16)], x1, mask=m)
            cnt = jnp.max(plsc.all_reduce_population_count(m))
            na = na + cnt
            return lax.cond(na >= PTH, drain, lambda v: v, na)

        def scan_block(b, na):
            off = b * SB
            c1 = pltpu.async_copy(srcp_h.at[pl.ds(off, SB)], eb.at[pl.ds(0, SB)], sem)
            c2 = pltpu.async_copy(dstp_h.at[pl.ds(off, SB)], eb.at[pl.ds(SB, SB)], sem)
            c3 = pltpu.async_copy(a0_h.at[pl.ds(off, SB)], eb.at[pl.ds(2 * SB, SB)], sem)
            c4 = pltpu.async_copy(a1_h.at[pl.ds(off, SB)], eb.at[pl.ds(3 * SB, SB)], sem)
            c1.wait(); c2.wait(); c3.wait(); c4.wait()
            return lax.fori_loop(0, SB // 16, sstep, na)

        na = lax.fori_loop(0, NSB, scan_block, 0)
        for h2 in range(2):
            relb[pl.ds(na + h2 * 16, 16)] = jnp.full((16,), RNG, jnp.int32)
            srcb[pl.ds(na + h2 * 16, 16)] = jnp.zeros((16,), jnp.int32)
            a0b[pl.ds(na + h2 * 16, 16)] = jnp.zeros((16,), jnp.float32)
            a1b[pl.ds(na + h2 * 16, 16)] = jnp.zeros((16,), jnp.float32)
        lax.fori_loop(0, (na + 31) // 32, dgroup, 0)
        pltpu.sync_copy(acc.at[pl.ds(0, RNG)],
                        out_h.at[pl.ds(rbase, RNG)])
        return carry

    lax.fori_loop(0, 2, pbody, 0)


def _sc_vector(hp_pad, srcf, dstf, a0, a1, zrows):
    return pl.kernel(
        _sc_vector_body,
        out_type=jax.ShapeDtypeStruct((NOUT, HID), jnp.float32),
        mesh=plsc.VectorSubcoreMesh(**_MESH),
        scratch_types=[pltpu.VMEM((4 * SB,), jnp.float32),
                       pltpu.VMEM((PCAP,), jnp.int32),
                       pltpu.VMEM((PCAP,), jnp.int32),
                       pltpu.VMEM((PCAP,), jnp.float32),
                       pltpu.VMEM((PCAP,), jnp.float32),
                       pltpu.VMEM((32, HID), jnp.float32),
                       pltpu.VMEM((RNG + 1, HID), jnp.float32),
                       pltpu.SemaphoreType.DMA],
        compiler_params=_SC_PARAMS,
    )(hp_pad, srcf, dstf, a0, a1, zrows)


def _edge_phase(hp, aux, srcp, dstp, zrows):
    auxp = jnp.pad(aux[:, :4], ((0, NP - N), (0, 0))).reshape(-1)
    hp_pad = jnp.pad(hp, ((0, NP - hp.shape[0]), (0, 0)))
    zden = jnp.zeros((DENW,), jnp.float32)
    ee0, ee1, den_part = _sc_scalar(auxp, srcp, dstp, zden)
    den = _den_reduce(den_part).reshape(-1)
    a0, a1 = _sc_alpha(ee0, ee1, dstp, den)
    srcf = lax.bitcast_convert_type(srcp, jnp.float32)
    dstf = lax.bitcast_convert_type(dstp, jnp.float32)
    return _sc_vector(hp_pad, srcf, dstf, a0, a1, zrows)


def kernel(x, edges, W1, b1, g1, be1, W2, b2, g2, be2, W3, b3, g3, be3,
           gW1, gas1, gad1, gb1, g4, be4, gW2, gas2, gad2, gb2, glg, glbe,
           pW1, pb1, pg1, pbe1, pW2, pb2, pg2, pbe2, pW3, pb3):
    loop = jnp.arange(N, dtype=edges.dtype)
    pad = jnp.full((EP - E,), N, dtype=edges.dtype)
    srcp = jnp.concatenate([edges[0], loop, pad])
    dstp = jnp.concatenate([edges[1], loop, pad])
    zrows = jnp.zeros((ZR, HID), jnp.float32)

    y1, s1 = _lin_stats(x, W1, b1)
    y2, s2 = _bn_lin(y1, s1, g1, be1, W2, b2)
    y3, s3 = _bn_lin(y2, s2, g2, be2, W3, b3)
    hp1, aux1 = _bn_lin_aux(y3, s3, g3, be3, gW1, gas1, gad1)
    acc1 = _edge_phase(hp1, aux1, srcp, dstp, zrows)
    t1, st1 = _post_gat(acc1, gb1)
    hp2, aux2 = _bn_lin_aux(t1, st1, g4, be4, gW2, gas2, gad2)
    acc2 = _edge_phase(hp2, aux2, srcp, dstp, zrows)
    t2, st2 = _post_gat(acc2, gb2)
    z1, sz1 = _bn_lin(t2, st2, glg, glbe, pW1, pb1)
    z2, sz2 = _bn_lin(z1, sz1, pg1, pbe1, pW2, pb2)
    return _final(z2, sz2, pg2, pbe2, pW3, pb3)
